# baseline (speedup 1.0000x reference)
import functools

import jax
import jax.numpy as jnp
from jax.experimental import pallas as pl
from jax.experimental.pallas import tpu as pltpu

N = 4096
B = 8
UNITS = 16
BU = B * UNITS
M = 3
CPAD = 32
F = B * CPAD
BLK = 512
NJB = N // BLK
KBLK = 1024
NJ = N // KBLK


def _build_body(a_ref, at_ref, amax_ref, dis_ref, acc_ref):
    j = pl.program_id(1)
    m = jnp.maximum(a_ref[...], at_ref[...].T)
    amax_ref[...] = m.astype(jnp.bfloat16)

    @pl.when(j == 0)
    def _():
        acc_ref[...] = jnp.zeros_like(acc_ref)

    acc_ref[...] += jnp.sum(m, axis=1, keepdims=True)

    @pl.when(j == NJB - 1)
    def _():
        d = acc_ref[...]
        dis_ref[...] = jnp.where(
            d > 0, jax.lax.rsqrt(jnp.maximum(d, 1e-12)), 0.0)


def _build(adj):
    return pl.pallas_call(
        _build_body,
        grid=(NJB, NJB),
        in_specs=[
            pl.BlockSpec((BLK, BLK), lambda i, j: (i, j)),
            pl.BlockSpec((BLK, BLK), lambda i, j: (j, i)),
        ],
        out_specs=[
            pl.BlockSpec((BLK, BLK), lambda i, j: (i, j)),
            pl.BlockSpec((BLK, 1), lambda i, j: (i, 0)),
        ],
        out_shape=[
            jax.ShapeDtypeStruct((N, N), jnp.bfloat16),
            jax.ShapeDtypeStruct((N, 1), jnp.float32),
        ],
        scratch_shapes=[pltpu.VMEM((BLK, 1), jnp.float32)],
    )(adj, adj)


def _inp_t_body(x_ref, o_ref):
    o_ref[...] = x_ref[...].T


def _inp_t(inputs):
    return pl.pallas_call(
        _inp_t_body,
        grid=(NJB,),
        in_specs=[pl.BlockSpec((B, BLK), lambda j: (0, j))],
        out_specs=pl.BlockSpec((BLK, B), lambda j: (j, 0)),
        out_shape=jax.ShapeDtypeStruct((N, B), jnp.float32),
    )(inputs)


def _h2n_body(h_ref, o_ref):
    o_ref[...] = jnp.concatenate([h_ref[b] for b in range(B)], axis=1)


def _h2n(h_bnu):
    return pl.pallas_call(
        _h2n_body,
        grid=(NJB,),
        in_specs=[pl.BlockSpec((B, BLK, UNITS), lambda j: (0, j, 0))],
        out_specs=pl.BlockSpec((BLK, BU), lambda j: (j, 0)),
        out_shape=jax.ShapeDtypeStruct((N, BU), jnp.float32),
    )(h_bnu)


def _n2b_body(x_ref, o_ref):
    for b in range(B):
        o_ref[b] = x_ref[:, b * UNITS:(b + 1) * UNITS]


def _n2b(x_n):
    return pl.pallas_call(
        _n2b_body,
        grid=(NJB,),
        in_specs=[pl.BlockSpec((BLK, BU), lambda j: (j, 0))],
        out_specs=pl.BlockSpec((B, BLK, UNITS), lambda j: (0, j, 0)),
        out_shape=jax.ShapeDtypeStruct((B, N, UNITS), jnp.float32),
    )(x_n)


def _diffuse(ci, a_ref, cur_ref, h_ref, disj_ref, disi_ref, x0s, x1s, acc):
    s = pl.program_id(0)
    j = pl.program_id(1)
    pad = CPAD - ci - UNITS

    @pl.when(s == 0)
    def _():
        pieces = []
        for b in range(B):
            sub = [cur_ref[:, b * ci:(b + 1) * ci],
                   h_ref[:, b * UNITS:(b + 1) * UNITS]]
            if pad:
                sub.append(jnp.zeros((KBLK, pad), jnp.float32))
            pieces.append(jnp.concatenate(sub, axis=1))
        x0v = jnp.concatenate(pieces, axis=1)
        x0s[pl.ds(j * KBLK, KBLK), :] = x0v
        xs = (x0v * disj_ref[...]).astype(jnp.bfloat16)
        part = jnp.dot(a_ref[...], xs, preferred_element_type=jnp.float32)

        @pl.when(j == 0)
        def _():
            acc[...] = part

        @pl.when(j > 0)
        def _():
            acc[...] += part

        @pl.when(j == NJ - 1)
        def _():
            x1s[...] = -disi_ref[...] * acc[...]

    @pl.when(s == 1)
    def _():
        xv = x1s[pl.ds(j * KBLK, KBLK), :]
        xs = (xv * disj_ref[...]).astype(jnp.bfloat16)
        part = jnp.dot(a_ref[...], xs, preferred_element_type=jnp.float32)

        @pl.when(j == 0)
        def _():
            acc[...] = part

        @pl.when(j > 0)
        def _():
            acc[...] += part


def _gate_body(ci, a_ref, cur_ref, h_ref, disj_ref, disi_ref,
               wr_ref, wu_ref, br_ref, bu_ref, hx_ref,
               rh_ref, u_ref, x0s, x1s, acc):
    _diffuse(ci, a_ref, cur_ref, h_ref, disj_ref, disi_ref, x0s, x1s, acc)

    @pl.when((pl.program_id(0) == 1) & (pl.program_id(1) == NJ - 1))
    def _():
        for i in range(NJ):
            sl = pl.ds(i * KBLK, KBLK)
            x0i = x0s[sl, :]
            x1i = x1s[sl, :]
            x2i = -2.0 * disi_ref[sl, :] * acc[sl, :] - x0i
            xs = (x0i, x1i, x2i)
            rps, ups = [], []
            for b in range(B):
                accr = br_ref[...]
                accu = bu_ref[...]
                for m in range(M):
                    xb = xs[m][:, b * CPAD:(b + 1) * CPAD]
                    accr = accr + jnp.dot(xb, wr_ref[m],
                                          preferred_element_type=jnp.float32)
                    accu = accu + jnp.dot(xb, wu_ref[m],
                                          preferred_element_type=jnp.float32)
                rps.append(jax.nn.sigmoid(accr))
                ups.append(jax.nn.sigmoid(accu))
            r_n = jnp.concatenate(rps, axis=1)
            u_n = jnp.concatenate(ups, axis=1)
            rh_ref[sl, :] = r_n * hx_ref[sl, :]
            u_ref[sl, :] = u_n


def _cand_body(ci, a_ref, cur_ref, h_ref, disj_ref, disi_ref,
               wc_ref, bc_ref, uin_ref, hx_ref,
               nh_ref, x0s, x1s, acc):
    _diffuse(ci, a_ref, cur_ref, h_ref, disj_ref, disi_ref, x0s, x1s, acc)

    @pl.when((pl.program_id(0) == 1) & (pl.program_id(1) == NJ - 1))
    def _():
        for i in range(NJ):
            sl = pl.ds(i * KBLK, KBLK)
            x0i = x0s[sl, :]
            x1i = x1s[sl, :]
            x2i = -2.0 * disi_ref[sl, :] * acc[sl, :] - x0i
            xs = (x0i, x1i, x2i)
            cps = []
            for b in range(B):
                acc_c = bc_ref[...]
                for m in range(M):
                    xb = xs[m][:, b * CPAD:(b + 1) * CPAD]
                    acc_c = acc_c + jnp.dot(
                        xb, wc_ref[m], preferred_element_type=jnp.float32)
                cps.append(jnp.tanh(acc_c))
            c_n = jnp.concatenate(cps, axis=1)
            u = uin_ref[sl, :]
            nh_ref[sl, :] = u * hx_ref[sl, :] + (1.0 - u) * c_n


def _common_specs(ci):
    return [
        pl.BlockSpec((N, KBLK), lambda s, j: (0, j)),
        pl.BlockSpec((KBLK, B * ci), lambda s, j: (j, 0)),
        pl.BlockSpec((KBLK, BU), lambda s, j: (j, 0)),
        pl.BlockSpec((KBLK, 1), lambda s, j: (j, 0)),
        pl.BlockSpec((N, 1), lambda s, j: (0, 0)),
    ]


_W_SPEC = pl.BlockSpec((M, CPAD, UNITS), lambda s, j: (0, 0, 0))
_B_SPEC = pl.BlockSpec((1, UNITS), lambda s, j: (0, 0))
_N_SPEC = pl.BlockSpec((N, BU), lambda s, j: (0, 0))
_N_SHAPE = jax.ShapeDtypeStruct((N, BU), jnp.float32)
_SCRATCH = [pltpu.VMEM((N, F), jnp.float32)] * 3


def _gate_gconv(amax, dis, cur_n, hx_n, wr, wu, br, bu, ci):
    return pl.pallas_call(
        functools.partial(_gate_body, ci),
        grid=(2, NJ),
        in_specs=_common_specs(ci) + [_W_SPEC, _W_SPEC, _B_SPEC, _B_SPEC,
                                      _N_SPEC],
        out_specs=[_N_SPEC, _N_SPEC],
        out_shape=[_N_SHAPE, _N_SHAPE],
        scratch_shapes=list(_SCRATCH),
    )(amax, cur_n, hx_n, dis, dis, wr, wu, br, bu, hx_n)


def _cand_gconv(amax, dis, cur_n, rh_n, wc, bc, u_n, hx_n, ci):
    return pl.pallas_call(
        functools.partial(_cand_body, ci),
        grid=(2, NJ),
        in_specs=_common_specs(ci) + [_W_SPEC, _B_SPEC, _N_SPEC, _N_SPEC],
        out_specs=_N_SPEC,
        out_shape=_N_SHAPE,
        scratch_shapes=list(_SCRATCH),
    )(amax, cur_n, rh_n, dis, dis, wc, bc, u_n, hx_n)


def _prep_w(W, C, O):
    Wr = jnp.transpose(W.reshape(C, M, O), (1, 0, 2))
    return jnp.pad(Wr, ((0, 0), (0, CPAD - C), (0, 0)))


def kernel(inputs, hidden_state, adj, W0_gate, b0_gate, W0_cand, b0_cand,
           W1_gate, b1_gate, W1_cand, b1_cand):
    amax, dis = _build(adj)

    params = [(W0_gate, b0_gate, W0_cand, b0_cand, 1),
              (W1_gate, b1_gate, W1_cand, b1_cand, UNITS)]
    cur_n = _inp_t(inputs)
    hs = []
    for l in range(2):
        Wg, bg, Wc, bc, ci = params[l]
        C = ci + UNITS
        wg = _prep_w(Wg, C, 2 * UNITS)
        wr, wu = wg[:, :, :UNITS], wg[:, :, UNITS:]
        br = bg[:UNITS].reshape(1, UNITS)
        bu = bg[UNITS:].reshape(1, UNITS)
        wc = _prep_w(Wc, C, UNITS)
        bcv = bc.reshape(1, UNITS)
        hx_n = _h2n(hidden_state[l].reshape(B, N, UNITS))

        rh_n, u_n = _gate_gconv(amax, dis, cur_n, hx_n, wr, wu, br, bu, ci)
        nh_n = _cand_gconv(amax, dis, cur_n, rh_n, wc, bcv, u_n, hx_n, ci)

        hs.append(_n2b(nh_n).reshape(B, N * UNITS))
        cur_n = nh_n

    return hs[-1], jnp.stack(hs, axis=0)


# speedup vs baseline: 1.1811x; 1.1811x over previous
import jax
import jax.numpy as jnp
from jax.experimental import pallas as pl
from jax.experimental.pallas import tpu as pltpu

N = 4096
B = 8
UNITS = 16
BU = B * UNITS
M = 3
CPAD = 32
F = B * CPAD
BLK = 512
NB = N // BLK
NJB = N // BLK
PBLK = 512
NP = N // PBLK


def _build_body(a_ref, at_ref, amax_ref, dis_ref, acc_ref):
    j = pl.program_id(1)
    m = jnp.maximum(a_ref[...], at_ref[...].T)
    amax_ref[...] = m.astype(jnp.bfloat16)

    @pl.when(j == 0)
    def _():
        acc_ref[...] = jnp.zeros_like(acc_ref)

    acc_ref[...] += jnp.sum(m, axis=1, keepdims=True)

    @pl.when(j == NJB - 1)
    def _():
        d = acc_ref[...]
        dis_ref[...] = jnp.where(
            d > 0, jax.lax.rsqrt(jnp.maximum(d, 1e-12)), 0.0)


def _build(adj):
    return pl.pallas_call(
        _build_body,
        grid=(NJB, NJB),
        in_specs=[
            pl.BlockSpec((BLK, BLK), lambda i, j: (i, j)),
            pl.BlockSpec((BLK, BLK), lambda i, j: (j, i)),
        ],
        out_specs=[
            pl.BlockSpec((BLK, BLK), lambda i, j: (i, j)),
            pl.BlockSpec((BLK, 1), lambda i, j: (i, 0)),
        ],
        out_shape=[
            jax.ShapeDtypeStruct((N, N), jnp.bfloat16),
            jax.ShapeDtypeStruct((N, 1), jnp.float32),
        ],
        scratch_shapes=[pltpu.VMEM((BLK, 1), jnp.float32)],
    )(adj, adj)


def _h2n_body(h_ref, o_ref):
    o_ref[...] = jnp.concatenate([h_ref[b] for b in range(B)], axis=1)


def _h2n(h_bnu):
    return pl.pallas_call(
        _h2n_body,
        grid=(NJB,),
        in_specs=[pl.BlockSpec((B, BLK, UNITS), lambda j: (0, j, 0))],
        out_specs=pl.BlockSpec((BLK, BU), lambda j: (j, 0)),
        out_shape=jax.ShapeDtypeStruct((N, BU), jnp.float32),
    )(h_bnu)


def _n2b_body(x_ref, o_ref):
    for b in range(B):
        o_ref[b] = x_ref[:, b * UNITS:(b + 1) * UNITS]


def _n2b(x_n):
    return pl.pallas_call(
        _n2b_body,
        grid=(NJB,),
        in_specs=[pl.BlockSpec((BLK, BU), lambda j: (j, 0))],
        out_specs=pl.BlockSpec((B, BLK, UNITS), lambda j: (0, j, 0)),
        out_shape=jax.ShapeDtypeStruct((B, N, UNITS), jnp.float32),
    )(x_n)


def _dcgru_body(a_ref, dis_ref, inp_ref, hx0_ref, hx1_ref,
                w0_ref, w12_ref, bias_ref,
                nh0_ref, nh1_ref, x0s, x1s, xs_s, rh_s, u_s):
    def assemble(get_cur, get_h, ci):
        pad = CPAD - ci - UNITS

        def body(i, _):
            lo = i * BLK
            sl = pl.ds(lo, BLK)
            curb = get_cur(sl)
            hb = get_h(sl)
            pieces = []
            for b in range(B):
                sub = [curb[:, b * ci:(b + 1) * ci],
                       hb[:, b * UNITS:(b + 1) * UNITS]]
                if pad:
                    sub.append(jnp.zeros((BLK, pad), jnp.float32))
                pieces.append(jnp.concatenate(sub, axis=1))
            x0s[sl, :] = jnp.concatenate(pieces, axis=1)
            return 0

        jax.lax.fori_loop(0, NB, body, 0)

    def scale_to_xs(src):
        def body(i, _):
            sl = pl.ds(i * BLK, BLK)
            xs_s[sl, :] = (src[sl, :] * dis_ref[sl, :]).astype(jnp.bfloat16)
            return 0

        jax.lax.fori_loop(0, NB, body, 0)

    def gconv(get_cur, get_h, ci, wi, gate):
        assemble(get_cur, get_h, ci)
        scale_to_xs(x0s)

        def x1_body(p, _):
            sl = pl.ds(p * PBLK, PBLK)
            part = jnp.dot(a_ref[sl, :], xs_s[...],
                           preferred_element_type=jnp.float32)
            x1s[sl, :] = -dis_ref[sl, :] * part
            return 0

        jax.lax.fori_loop(0, NP, x1_body, 0)
        scale_to_xs(x1s)

        def x2_body(p, _):
            sl = pl.ds(p * PBLK, PBLK)
            part = jnp.dot(a_ref[sl, :], xs_s[...],
                           preferred_element_type=jnp.float32)
            x2v = -2.0 * dis_ref[sl, :] * part - x0s[sl, :]
            x2b = x2v.astype(jnp.bfloat16)

            def cmb(k):
                acc = bias_ref[k][None, :]
                acc = acc + jnp.dot(x0s[sl, :], w0_ref[k],
                                    preferred_element_type=jnp.float32)
                acc = acc + jnp.dot(x1s[sl, :].astype(jnp.bfloat16),
                                    w12_ref[k, 0],
                                    preferred_element_type=jnp.float32)
                acc = acc + jnp.dot(x2b, w12_ref[k, 1],
                                    preferred_element_type=jnp.float32)
                return acc

            hxv = (hx0_ref if wi == 0 else hx1_ref)[sl, :]
            if gate:
                r = jax.nn.sigmoid(cmb(3 * wi))
                rh_s[sl, :] = r * hxv
                u_s[sl, :] = jax.nn.sigmoid(cmb(3 * wi + 1))
            else:
                c = jnp.tanh(cmb(3 * wi + 2))
                u = u_s[sl, :]
                nh = nh0_ref if wi == 0 else nh1_ref
                nh[sl, :] = u * hxv + (1.0 - u) * c
            return 0

        jax.lax.fori_loop(0, NP, x2_body, 0)

    def cur0(sl):
        return inp_ref[:, sl].T

    def hx0(sl):
        return hx0_ref[sl, :]

    def rh(sl):
        return rh_s[sl, :]

    gconv(cur0, hx0, 1, 0, True)
    gconv(cur0, rh, 1, 0, False)

    def cur1(sl):
        return nh0_ref[sl, :]

    def hx1(sl):
        return hx1_ref[sl, :]

    gconv(cur1, hx1, UNITS, 1, True)
    gconv(cur1, rh, UNITS, 1, False)


def _dcgru(amax, dis, inputs, hx0_n, hx1_n, w0, w12, bias):
    return pl.pallas_call(
        _dcgru_body,
        compiler_params=pltpu.CompilerParams(
            vmem_limit_bytes=100 * 1024 * 1024),
        out_shape=[
            jax.ShapeDtypeStruct((N, BU), jnp.float32),
            jax.ShapeDtypeStruct((N, BU), jnp.float32),
        ],
        scratch_shapes=[
            pltpu.VMEM((N, F), jnp.float32),
            pltpu.VMEM((N, F), jnp.float32),
            pltpu.VMEM((N, F), jnp.bfloat16),
            pltpu.VMEM((N, BU), jnp.float32),
            pltpu.VMEM((N, BU), jnp.float32),
        ],
    )(amax, dis, inputs, hx0_n, hx1_n, w0, w12, bias)


def _prep_w(W, C, O):
    Wr = jnp.transpose(W.reshape(C, M, O), (1, 0, 2))
    return jnp.pad(Wr, ((0, 0), (0, CPAD - C), (0, 0)))


def kernel(inputs, hidden_state, adj, W0_gate, b0_gate, W0_cand, b0_cand,
           W1_gate, b1_gate, W1_cand, b1_cand):
    amax, dis = _build(adj)

    eye = jnp.eye(B, dtype=jnp.float32)
    w0_list, w12_list, b_list = [], [], []
    for (Wg, bg, Wc, bc, C) in [
        (W0_gate, b0_gate, W0_cand, b0_cand, 1 + UNITS),
        (W1_gate, b1_gate, W1_cand, b1_cand, 2 * UNITS),
    ]:
        wg = _prep_w(Wg, C, 2 * UNITS)
        wc = _prep_w(Wc, C, UNITS)
        for wm, bv in ((wg[:, :, :UNITS], bg[:UNITS]),
                       (wg[:, :, UNITS:], bg[UNITS:]),
                       (wc, bc)):
            bd = jnp.stack([jnp.kron(eye, wm[m]) for m in range(M)])
            w0_list.append(bd[0])
            w12_list.append(bd[1:])
            b_list.append(jnp.tile(bv, B))
    w0 = jnp.stack(w0_list)
    w12 = jnp.stack(w12_list).astype(jnp.bfloat16)
    bias = jnp.stack(b_list)

    hx0_n = _h2n(hidden_state[0].reshape(B, N, UNITS))
    hx1_n = _h2n(hidden_state[1].reshape(B, N, UNITS))

    nh0_n, nh1_n = _dcgru(amax, dis, inputs, hx0_n, hx1_n, w0, w12, bias)

    h0 = _n2b(nh0_n).reshape(B, N * UNITS)
    h1 = _n2b(nh1_n).reshape(B, N * UNITS)
    return h1, jnp.stack([h0, h1], axis=0)
